# Initial kernel scaffold; baseline (speedup 1.0000x reference)
import jax
import jax.numpy as jnp
from jax.experimental import pallas as pl


def kernel(x, pos_table):
    raise NotImplementedError("write your pallas kernel here")


# speedup vs baseline: 3.5303x; 3.5303x over previous
import jax
import jax.numpy as jnp
from jax.experimental import pallas as pl


def _add_kernel(x_ref, pos_ref, o_ref):
    o_ref[...] = x_ref[...] + pos_ref[...]


def kernel(x, pos_table):
    B, P, E = x.shape
    return pl.pallas_call(
        _add_kernel,
        grid=(B,),
        in_specs=[
            pl.BlockSpec((1, P, E), lambda b: (b, 0, 0)),
            pl.BlockSpec((P, E), lambda b: (0, 0)),
        ],
        out_specs=pl.BlockSpec((1, P, E), lambda b: (b, 0, 0)),
        out_shape=jax.ShapeDtypeStruct((B, P, E), x.dtype),
    )(x, pos_table)
